# baseline (speedup 1.0000x reference)
import jax
import jax.numpy as jnp
from jax import lax
from jax.experimental import pallas as pl
from jax.experimental.pallas import tpu as pltpu
from jax.experimental.pallas import tpu_sc as plsc

VOCAB_ROWS = 1001
D = 64
BATCH = 4096
HIST = 200
NC, NS = 2, 16
NW = NC * NS
N_WT = 8
N_WB = 4
T_PER_W = HIST // N_WT
N_TT = HIST // 8
TT_PER_W = 4
N_U = T_PER_W * 8


def _emb_body(table_hbm, tok_hbm, out_hbm, table_v, tok_v, buf0, buf1,
              osem0, osem1):
    wid = lax.axis_index("s") * NC + lax.axis_index("c")
    wt = wid // N_WB
    wb = wid % N_WB
    t0 = wt * T_PER_W
    tt0 = t0 // 8

    pltpu.sync_copy(table_hbm, table_v)
    pltpu.sync_copy(
        tok_hbm.at[pl.ds(tt0, TT_PER_W), pl.ds(wb * 8, 8)], tok_v)

    bufs = (buf0, buf1)
    osems = (osem0, osem1)

    def compute(tl, dt, buf):
        t = t0 + tl
        ttl = t // 8 - tt0
        ti = t % 8
        dbase = dt * 8
        dvecs = [jnp.full((16,), (dbase + di) * VOCAB_ROWS, jnp.int32)
                 for di in range(8)]

        @plsc.parallel_loop(0, 64, unroll=16)
        def _(g):
            tokv = tok_v[ttl, g // 8, ti, pl.ds((g % 8) * 16, 16)]
            for di in range(8):
                v = plsc.load_gather(table_v, [tokv + dvecs[di]])
                buf[g // 8, di, pl.ds((g % 8) * 16, 16)] = v

    def dst(u):
        return out_hbm.at[t0 + u // 8, u % 8, pl.ds(wb * 8, 8)]

    def body(u2, carry):
        for p in range(2):
            u = u2 * 2 + p

            @pl.when(u2 > 0)
            def _():
                pltpu.make_async_copy(bufs[p], dst(u), osems[p]).wait()

            compute(u // 8, u % 8, bufs[p])
            pltpu.async_copy(bufs[p], dst(u), osems[p])
        return carry

    lax.fori_loop(0, N_U // 2, body, 0)
    pltpu.make_async_copy(bufs[0], dst(N_U - 2), osems[0]).wait()
    pltpu.make_async_copy(bufs[1], dst(N_U - 1), osems[1]).wait()


def kernel(table, tokens):
    table_flat = table.T.reshape(D * VOCAB_ROWS)
    tok_native = (tokens.T.reshape(N_TT, 8, NW, 128)
                  .transpose(0, 2, 1, 3))
    mesh = plsc.VectorSubcoreMesh(core_axis_name="c", subcore_axis_name="s")
    out5 = pl.kernel(
        _emb_body,
        mesh=mesh,
        compiler_params=pltpu.CompilerParams(use_tc_tiling_on_sc=False,
                                             needs_layout_passes=False),
        out_type=jax.ShapeDtypeStruct((HIST, 8, NW, 8, 128), jnp.float32),
        scratch_types=[
            pltpu.VMEM((D * VOCAB_ROWS,), jnp.float32),
            pltpu.VMEM((TT_PER_W, 8, 8, 128), jnp.int32),
            pltpu.VMEM((8, 8, 128), jnp.float32),
            pltpu.VMEM((8, 8, 128), jnp.float32),
            pltpu.SemaphoreType.DMA,
            pltpu.SemaphoreType.DMA,
        ],
    )(table_flat, tok_native)
    return out5.transpose(2, 4, 0, 1, 3).reshape(BATCH, HIST, D)


# speedup vs baseline: 1.1960x; 1.1960x over previous
import jax
import jax.numpy as jnp
from jax import lax
from jax.experimental import pallas as pl
from jax.experimental.pallas import tpu as pltpu
from jax.experimental.pallas import tpu_sc as plsc

VOCAB_ROWS = 1001
D = 64
BATCH = 4096
HIST = 200
NC, NS = 2, 16
NW = NC * NS
N_WT = 8
N_WB = 4
T_PER_W = HIST // N_WT
N_TT = HIST // 8
TT_PER_W = 4
N_U = T_PER_W * 8


def _emb_body(table_hbm, tok_hbm, out_hbm, table_v, tok_v, buf0, buf1,
              osem0, osem1):
    wid = lax.axis_index("s") * NC + lax.axis_index("c")
    wt = wid // N_WB
    wb = wid % N_WB
    t0 = wt * T_PER_W
    tt0 = t0 // 8

    pltpu.sync_copy(table_hbm, table_v)
    pltpu.sync_copy(
        tok_hbm.at[pl.ds(tt0, TT_PER_W), pl.ds(wb * 8, 8)], tok_v)

    bufs = (buf0, buf1)
    osems = (osem0, osem1)

    def compute(tl, dt, buf):
        t = t0 + tl
        ttl = t // 8 - tt0
        ti = t % 8
        dbase = dt * 8
        dvecs = [jnp.full((16,), (dbase + di) * VOCAB_ROWS, jnp.int32)
                 for di in range(8)]

        @plsc.parallel_loop(0, 64, unroll=8)
        def _(g):
            tokv = tok_v[ttl, g // 8, ti, pl.ds((g % 8) * 16, 16)]
            for di in range(8):
                v = plsc.load_gather(table_v, [tokv + dvecs[di]])
                buf[g // 8, di, pl.ds((g % 8) * 16, 16)] = v

    def dst(u):
        return out_hbm.at[t0 + u // 8, u % 8, pl.ds(wb * 8, 8)]

    def body(u2, carry):
        for p in range(2):
            u = u2 * 2 + p

            @pl.when(u2 > 0)
            def _():
                pltpu.make_async_copy(bufs[p], dst(u), osems[p]).wait()

            compute(u // 8, u % 8, bufs[p])
            pltpu.async_copy(bufs[p], dst(u), osems[p])
        return carry

    lax.fori_loop(0, N_U // 2, body, 0)
    pltpu.make_async_copy(bufs[0], dst(N_U - 2), osems[0]).wait()
    pltpu.make_async_copy(bufs[1], dst(N_U - 1), osems[1]).wait()


def kernel(table, tokens):
    table_flat = table.T.reshape(D * VOCAB_ROWS)
    tok_native = (tokens.T.reshape(N_TT, 8, NW, 128)
                  .transpose(0, 2, 1, 3))
    mesh = plsc.VectorSubcoreMesh(core_axis_name="c", subcore_axis_name="s")
    out5 = pl.kernel(
        _emb_body,
        mesh=mesh,
        compiler_params=pltpu.CompilerParams(use_tc_tiling_on_sc=False,
                                             needs_layout_passes=False),
        out_type=jax.ShapeDtypeStruct((HIST, 8, NW, 8, 128), jnp.float32),
        scratch_types=[
            pltpu.VMEM((D * VOCAB_ROWS,), jnp.float32),
            pltpu.VMEM((TT_PER_W, 8, 8, 128), jnp.int32),
            pltpu.VMEM((8, 8, 128), jnp.float32),
            pltpu.VMEM((8, 8, 128), jnp.float32),
            pltpu.SemaphoreType.DMA,
            pltpu.SemaphoreType.DMA,
        ],
    )(table_flat, tok_native)
    return out5.transpose(2, 4, 0, 1, 3).reshape(BATCH, HIST, D)
